# Initial kernel scaffold; baseline (speedup 1.0000x reference)
import jax
import jax.numpy as jnp
from jax.experimental import pallas as pl


def kernel(x, edge_index, W1, b1, W2, b2, W3, b3, Wc1, bc1, Wc2, bc2):
    raise NotImplementedError("write your pallas kernel here")


# speedup vs baseline: 16.6089x; 16.6089x over previous
import functools

import jax
import jax.numpy as jnp
from jax import lax
from jax.experimental import pallas as pl
from jax.experimental.pallas import tpu as pltpu
from jax.experimental.pallas import tpu_sc as plsc

_NC = 2
_NS = 16
_CHUNK = 128


def _sc_edge_accumulate(hp, src, dst):
    N, H = hp.shape
    E = src.shape[0]
    nw = _NC * _NS
    per_w = E // nw
    n_full, tail = divmod(per_w, _CHUNK)
    rps = N // _NS

    mesh = plsc.VectorSubcoreMesh(core_axis_name="c", subcore_axis_name="s")

    @functools.partial(
        pl.kernel,
        mesh=mesh,
        out_type=jax.ShapeDtypeStruct((2 * N, H), jnp.float32),
        compiler_params=pltpu.CompilerParams(use_tc_tiling_on_sc=False),
        scratch_types=[
            pltpu.VMEM((_CHUNK,), jnp.int32),
            pltpu.VMEM((_CHUNK,), jnp.int32),
            pltpu.VMEM((_CHUNK, H), jnp.float32),
            pltpu.VMEM((max(tail, 8),), jnp.int32),
            pltpu.VMEM((max(tail, 8),), jnp.int32),
            pltpu.VMEM((max(tail, 8), H), jnp.float32),
            pltpu.VMEM((rps, H), jnp.float32),
            pltpu.VMEM_SHARED((N, H), jnp.float32),
            pltpu.SemaphoreType.DMA,
        ],
    )
    def ker(hp_hbm, src_hbm, dst_hbm, out_hbm,
            sidx, didx, rows, sidx_t, didx_t, rows_t, zbuf, acc, sem):
        c = lax.axis_index("c")
        s = lax.axis_index("s")
        zero16 = jnp.zeros((16,), jnp.float32)

        def zrow(i, carry):
            for j in range(H // 16):
                zbuf[i, pl.ds(j * 16, 16)] = zero16
            return carry

        lax.fori_loop(0, rps, zrow, 0)
        pltpu.sync_copy(zbuf, acc.at[pl.ds(s * rps, rps)])
        plsc.subcore_barrier()

        eb = c * (E // _NC) + s * per_w

        def body(i, carry):
            base = eb + i * _CHUNK
            pltpu.sync_copy(src_hbm.at[pl.ds(base, _CHUNK)], sidx)
            pltpu.sync_copy(dst_hbm.at[pl.ds(base, _CHUNK)], didx)
            pltpu.async_copy(hp_hbm.at[sidx], rows, sem).wait()
            pltpu.sync_copy(rows, acc.at[didx], add=True)
            return carry

        lax.fori_loop(0, n_full, body, 0)
        if tail:
            base = eb + n_full * _CHUNK
            pltpu.sync_copy(src_hbm.at[pl.ds(base, tail)], sidx_t)
            pltpu.sync_copy(dst_hbm.at[pl.ds(base, tail)], didx_t)
            pltpu.async_copy(hp_hbm.at[sidx_t], rows_t, sem).wait()
            pltpu.sync_copy(rows_t, acc.at[didx_t], add=True)
        plsc.subcore_barrier()

        pltpu.sync_copy(acc.at[pl.ds(s * rps, rps)],
                        out_hbm.at[pl.ds(c * N + s * rps, rps)])

    return ker(hp, src, dst)


def _sc_degree(dst, n_nodes):
    N = n_nodes
    W = 16
    E = dst.shape[0]
    nw = _NC * _NS
    per_w = E // nw
    n_full, tail = divmod(per_w, _CHUNK)
    rps = N // _NS

    mesh = plsc.VectorSubcoreMesh(core_axis_name="c", subcore_axis_name="s")

    @functools.partial(
        pl.kernel,
        mesh=mesh,
        out_type=jax.ShapeDtypeStruct((2 * N, W), jnp.float32),
        compiler_params=pltpu.CompilerParams(use_tc_tiling_on_sc=False),
        scratch_types=[
            pltpu.VMEM((_CHUNK,), jnp.int32),
            pltpu.VMEM((max(tail, 8),), jnp.int32),
            pltpu.VMEM((_CHUNK, W), jnp.float32),
            pltpu.VMEM((max(tail, 8), W), jnp.float32),
            pltpu.VMEM((rps, W), jnp.float32),
            pltpu.VMEM_SHARED((N, W), jnp.float32),
        ],
    )
    def ker(dst_hbm, out_hbm, didx, didx_t, ones, ones_t, zbuf, acc):
        c = lax.axis_index("c")
        s = lax.axis_index("s")
        onehot = jnp.where(lax.iota(jnp.int32, 16) == 0,
                           jnp.float32(1), jnp.float32(0))
        zero16 = jnp.zeros((16,), jnp.float32)

        def fill(i, carry):
            ones[i, :] = onehot
            return carry

        lax.fori_loop(0, _CHUNK, fill, 0)

        def fill_t(i, carry):
            ones_t[i, :] = onehot
            return carry

        lax.fori_loop(0, max(tail, 8), fill_t, 0)

        def zrow(i, carry):
            zbuf[i, :] = zero16
            return carry

        lax.fori_loop(0, rps, zrow, 0)
        pltpu.sync_copy(zbuf, acc.at[pl.ds(s * rps, rps)])
        plsc.subcore_barrier()

        eb = c * (E // _NC) + s * per_w

        def body(i, carry):
            base = eb + i * _CHUNK
            pltpu.sync_copy(dst_hbm.at[pl.ds(base, _CHUNK)], didx)
            pltpu.sync_copy(ones, acc.at[didx], add=True)
            return carry

        lax.fori_loop(0, n_full, body, 0)
        if tail:
            base = eb + n_full * _CHUNK
            pltpu.sync_copy(dst_hbm.at[pl.ds(base, tail)], didx_t)
            pltpu.sync_copy(ones_t, acc.at[didx_t], add=True)
        plsc.subcore_barrier()

        pltpu.sync_copy(acc.at[pl.ds(s * rps, rps)],
                        out_hbm.at[pl.ds(c * N + s * rps, rps)])

    return ker(dst)


def _tc_first(degp, x, W1):
    N = x.shape[0]
    H = W1.shape[1]

    def body(deg_ref, x_ref, w_ref, hp_ref, dinv_ref):
        deg = deg_ref[0:N, 0:1] + deg_ref[N:2 * N, 0:1] + 1.0
        dinv = lax.rsqrt(deg)
        dinv_ref[...] = dinv
        hp_ref[...] = jnp.dot(x_ref[...], w_ref[...],
                              preferred_element_type=jnp.float32) * dinv

    return pl.pallas_call(
        body,
        out_shape=(jax.ShapeDtypeStruct((N, H), jnp.float32),
                   jax.ShapeDtypeStruct((N, 1), jnp.float32)),
    )(degp, x, W1)


def _tc_mid(e, hp, dinv, b, W):
    N, H = hp.shape
    HO = W.shape[1]

    def body(e_ref, hp_ref, dinv_ref, b_ref, w_ref, out_ref):
        esum = e_ref[0:N, :] + e_ref[N:2 * N, :] + hp_ref[...]
        h = jnp.maximum(esum * dinv_ref[...] + b_ref[...], 0.0)
        out_ref[...] = jnp.dot(h, w_ref[...],
                               preferred_element_type=jnp.float32) * dinv_ref[...]

    return pl.pallas_call(
        body,
        out_shape=jax.ShapeDtypeStruct((N, HO), jnp.float32),
    )(e, hp, dinv, b, W)


def _tc_final(e, hp, dinv, b3, Wc1, bc1, Wc2, bc2):
    N, H = hp.shape

    def body(e_ref, hp_ref, dinv_ref, b3_ref, wc1_ref, bc1_ref, wc2_ref,
             bc2_ref, out_ref):
        esum = e_ref[0:N, :] + e_ref[N:2 * N, :] + hp_ref[...]
        h3 = esum * dinv_ref[...] + b3_ref[...]
        g = jnp.sum(h3, axis=0, keepdims=True) * jnp.float32(1.0 / N)
        z = jnp.maximum(jnp.dot(g, wc1_ref[...],
                                preferred_element_type=jnp.float32)
                        + bc1_ref[...], 0.0)
        out_ref[...] = jnp.dot(z, wc2_ref[...],
                               preferred_element_type=jnp.float32) + bc2_ref[...]

    return pl.pallas_call(
        body,
        out_shape=jax.ShapeDtypeStruct((1, Wc2.shape[1]), jnp.float32),
    )(e, hp, dinv, b3, Wc1, bc1, Wc2, bc2)


def kernel(x, edge_index, W1, b1, W2, b2, W3, b3, Wc1, bc1, Wc2, bc2):
    N = x.shape[0]
    src = edge_index[0]
    dst = edge_index[1]

    degp = _sc_degree(dst, N)
    h1p, dinv = _tc_first(degp, x, W1)

    e1 = _sc_edge_accumulate(h1p, src, dst)
    h2p = _tc_mid(e1, h1p, dinv, b1.reshape(1, -1), W2)

    e2 = _sc_edge_accumulate(h2p, src, dst)
    h3p = _tc_mid(e2, h2p, dinv, b2.reshape(1, -1), W3)

    e3 = _sc_edge_accumulate(h3p, src, dst)
    out = _tc_final(e3, h3p, dinv, b3.reshape(1, -1), Wc1,
                    bc1.reshape(1, -1), Wc2, bc2.reshape(1, -1))
    return out


# speedup vs baseline: 34.4819x; 2.0761x over previous
import functools

import jax
import jax.numpy as jnp
from jax import lax
from jax.experimental import pallas as pl
from jax.experimental.pallas import tpu as pltpu
from jax.experimental.pallas import tpu_sc as plsc

_NC = 2
_NS = 16
_CHUNK = 100
_NBUF = 4


def _sc_edge_accumulate(hp, src2, dst2):
    N, H = hp.shape
    n_rows = src2.shape[0]
    nw = _NC * _NS
    rw = n_rows // nw
    n_grp = rw // _NBUF
    rps = N // _NS

    mesh = plsc.VectorSubcoreMesh(core_axis_name="c", subcore_axis_name="s")

    @functools.partial(
        pl.kernel,
        mesh=mesh,
        out_type=jax.ShapeDtypeStruct((2 * N, H), jnp.float32),
        compiler_params=pltpu.CompilerParams(use_tc_tiling_on_sc=False),
        scratch_types=[
            pltpu.VMEM((rw, _CHUNK), jnp.int32),
            pltpu.VMEM((rw, _CHUNK), jnp.int32),
            [pltpu.VMEM((_CHUNK, H), jnp.float32)] * _NBUF,
            pltpu.VMEM((rps, H), jnp.float32),
            pltpu.VMEM_SHARED((N, H), jnp.float32),
            pltpu.SemaphoreType.DMA,
        ],
    )
    def ker(hp_hbm, src_hbm, dst_hbm, out_hbm,
            sidx, didx, rows, zbuf, acc, sem):
        c = lax.axis_index("c")
        s = lax.axis_index("s")
        w = c * _NS + s
        zero16 = jnp.zeros((16,), jnp.float32)

        def zrow(i, carry):
            for j in range(H // 16):
                zbuf[i, pl.ds(j * 16, 16)] = zero16
            return carry

        lax.fori_loop(0, rps, zrow, 0)
        pltpu.sync_copy(zbuf, acc.at[pl.ds(s * rps, rps)])
        pltpu.sync_copy(src_hbm.at[pl.ds(w * rw, rw)], sidx)
        pltpu.sync_copy(dst_hbm.at[pl.ds(w * rw, rw)], didx)
        plsc.subcore_barrier()

        def group(g, carry):
            jb = g * _NBUF
            handles = [
                pltpu.async_copy(hp_hbm.at[sidx.at[jb + b]], rows[b], sem)
                for b in range(_NBUF)
            ]
            for b in range(_NBUF):
                handles[b].wait()
                pltpu.sync_copy(rows[b], acc.at[didx.at[jb + b]], add=True)
            return carry

        lax.fori_loop(0, n_grp, group, 0)
        plsc.subcore_barrier()

        pltpu.sync_copy(acc.at[pl.ds(s * rps, rps)],
                        out_hbm.at[pl.ds(c * N + s * rps, rps)])

    return ker(hp, src2, dst2)


def _sc_degree(dst2, n_nodes):
    N = n_nodes
    W = 16
    n_rows = dst2.shape[0]
    nw = _NC * _NS
    rw = n_rows // nw
    rps = N // _NS

    mesh = plsc.VectorSubcoreMesh(core_axis_name="c", subcore_axis_name="s")

    @functools.partial(
        pl.kernel,
        mesh=mesh,
        out_type=jax.ShapeDtypeStruct((2 * N, W), jnp.float32),
        compiler_params=pltpu.CompilerParams(use_tc_tiling_on_sc=False),
        scratch_types=[
            pltpu.VMEM((rw, _CHUNK), jnp.int32),
            pltpu.VMEM((_CHUNK, W), jnp.float32),
            pltpu.VMEM((rps, W), jnp.float32),
            pltpu.VMEM_SHARED((N, W), jnp.float32),
        ],
    )
    def ker(dst_hbm, out_hbm, didx, ones, zbuf, acc):
        c = lax.axis_index("c")
        s = lax.axis_index("s")
        w = c * _NS + s
        onehot = jnp.where(lax.iota(jnp.int32, 16) == 0,
                           jnp.float32(1), jnp.float32(0))
        zero16 = jnp.zeros((16,), jnp.float32)

        def fill(i, carry):
            ones[i, :] = onehot
            return carry

        lax.fori_loop(0, _CHUNK, fill, 0)

        def zrow(i, carry):
            zbuf[i, :] = zero16
            return carry

        lax.fori_loop(0, rps, zrow, 0)
        pltpu.sync_copy(zbuf, acc.at[pl.ds(s * rps, rps)])
        pltpu.sync_copy(dst_hbm.at[pl.ds(w * rw, rw)], didx)
        plsc.subcore_barrier()

        def body(j, carry):
            pltpu.sync_copy(ones, acc.at[didx.at[j]], add=True)
            return carry

        lax.fori_loop(0, rw, body, 0)
        plsc.subcore_barrier()

        pltpu.sync_copy(acc.at[pl.ds(s * rps, rps)],
                        out_hbm.at[pl.ds(c * N + s * rps, rps)])

    return ker(dst2)


def _tc_first(degp, x, W1):
    N = x.shape[0]
    H = W1.shape[1]

    def body(deg_ref, x_ref, w_ref, hp_ref, dinv_ref):
        deg = deg_ref[0:N, 0:1] + deg_ref[N:2 * N, 0:1] + 1.0
        dinv = lax.rsqrt(deg)
        dinv_ref[...] = dinv
        hp_ref[...] = jnp.dot(x_ref[...], w_ref[...],
                              preferred_element_type=jnp.float32) * dinv

    return pl.pallas_call(
        body,
        out_shape=(jax.ShapeDtypeStruct((N, H), jnp.float32),
                   jax.ShapeDtypeStruct((N, 1), jnp.float32)),
    )(degp, x, W1)


def _tc_mid(e, hp, dinv, b, W):
    N, H = hp.shape
    HO = W.shape[1]

    def body(e_ref, hp_ref, dinv_ref, b_ref, w_ref, out_ref):
        esum = e_ref[0:N, :] + e_ref[N:2 * N, :] + hp_ref[...]
        h = jnp.maximum(esum * dinv_ref[...] + b_ref[...], 0.0)
        out_ref[...] = jnp.dot(h, w_ref[...],
                               preferred_element_type=jnp.float32) * dinv_ref[...]

    return pl.pallas_call(
        body,
        out_shape=jax.ShapeDtypeStruct((N, HO), jnp.float32),
    )(e, hp, dinv, b, W)


def _tc_final(e, hp, dinv, b3, Wc1, bc1, Wc2, bc2):
    N, H = hp.shape

    def body(e_ref, hp_ref, dinv_ref, b3_ref, wc1_ref, bc1_ref, wc2_ref,
             bc2_ref, out_ref):
        esum = e_ref[0:N, :] + e_ref[N:2 * N, :] + hp_ref[...]
        h3 = esum * dinv_ref[...] + b3_ref[...]
        g = jnp.sum(h3, axis=0, keepdims=True) * jnp.float32(1.0 / N)
        z = jnp.maximum(jnp.dot(g, wc1_ref[...],
                                preferred_element_type=jnp.float32)
                        + bc1_ref[...], 0.0)
        out_ref[...] = jnp.dot(z, wc2_ref[...],
                               preferred_element_type=jnp.float32) + bc2_ref[...]

    return pl.pallas_call(
        body,
        out_shape=jax.ShapeDtypeStruct((1, Wc2.shape[1]), jnp.float32),
    )(e, hp, dinv, b3, Wc1, bc1, Wc2, bc2)


def kernel(x, edge_index, W1, b1, W2, b2, W3, b3, Wc1, bc1, Wc2, bc2):
    N = x.shape[0]
    src2 = edge_index[0].reshape(-1, _CHUNK)
    dst2 = edge_index[1].reshape(-1, _CHUNK)

    degp = _sc_degree(dst2, N)
    h1p, dinv = _tc_first(degp, x, W1)

    e1 = _sc_edge_accumulate(h1p, src2, dst2)
    h2p = _tc_mid(e1, h1p, dinv, b1.reshape(1, -1), W2)

    e2 = _sc_edge_accumulate(h2p, src2, dst2)
    h3p = _tc_mid(e2, h2p, dinv, b2.reshape(1, -1), W3)

    e3 = _sc_edge_accumulate(h3p, src2, dst2)
    out = _tc_final(e3, h3p, dinv, b3.reshape(1, -1), Wc1,
                    bc1.reshape(1, -1), Wc2, bc2.reshape(1, -1))
    return out


# speedup vs baseline: 37.4887x; 1.0872x over previous
import functools

import jax
import jax.numpy as jnp
from jax import lax
from jax.experimental import pallas as pl
from jax.experimental.pallas import tpu as pltpu
from jax.experimental.pallas import tpu_sc as plsc

_NC = 2
_NS = 16
_CHUNK = 100
_NBUF = 10
_ZROWS = 125


def _sc_edge_accumulate(hp, src2, dst2):
    N, H = hp.shape
    n_rows = src2.shape[0]
    nw = _NC * _NS
    rw = n_rows // nw
    n_grp = rw // _NBUF
    rps = N // _NS

    mesh = plsc.VectorSubcoreMesh(core_axis_name="c", subcore_axis_name="s")

    @functools.partial(
        pl.kernel,
        mesh=mesh,
        out_type=jax.ShapeDtypeStruct((2 * N, H), jnp.float32),
        compiler_params=pltpu.CompilerParams(use_tc_tiling_on_sc=False),
        scratch_types=[
            pltpu.VMEM((rw, _CHUNK), jnp.int32),
            pltpu.VMEM((rw, _CHUNK), jnp.int32),
            [pltpu.VMEM((_CHUNK, H), jnp.float32)] * _NBUF,
            pltpu.VMEM_SHARED((N, H), jnp.float32),
            pltpu.SemaphoreType.DMA,
        ],
    )
    def ker(hp_hbm, src_hbm, dst_hbm, out_hbm,
            sidx, didx, rows, acc, sem):
        c = lax.axis_index("c")
        s = lax.axis_index("s")
        w = c * _NS + s
        zero16 = jnp.zeros((16,), jnp.float32)

        def zrow(i, carry):
            for j in range(H // 16):
                rows[0][i, pl.ds(j * 16, 16)] = zero16
            return carry

        lax.fori_loop(0, _CHUNK, zrow, 0)
        zfull, zrem = divmod(rps, _CHUNK)
        for z in range(zfull):
            pltpu.sync_copy(rows[0], acc.at[pl.ds(s * rps + z * _CHUNK, _CHUNK)])
        if zrem:
            pltpu.sync_copy(rows[0].at[pl.ds(0, zrem)],
                            acc.at[pl.ds(s * rps + zfull * _CHUNK, zrem)])
        pltpu.sync_copy(src_hbm.at[pl.ds(w * rw, rw)], sidx)
        pltpu.sync_copy(dst_hbm.at[pl.ds(w * rw, rw)], didx)
        plsc.subcore_barrier()

        def group(g, carry):
            jb = g * _NBUF
            handles = [
                pltpu.async_copy(hp_hbm.at[sidx.at[jb + b]], rows[b], sem)
                for b in range(_NBUF)
            ]
            for b in range(_NBUF):
                handles[b].wait()
                pltpu.sync_copy(rows[b], acc.at[didx.at[jb + b]], add=True)
            return carry

        lax.fori_loop(0, n_grp, group, 0)
        plsc.subcore_barrier()

        pltpu.sync_copy(acc.at[pl.ds(s * rps, rps)],
                        out_hbm.at[pl.ds(c * N + s * rps, rps)])

    return ker(hp, src2, dst2)


def _sc_degree(dst2, n_nodes):
    N = n_nodes
    W = 16
    n_rows = dst2.shape[0]
    nw = _NC * _NS
    rw = n_rows // nw
    rps = N // _NS

    mesh = plsc.VectorSubcoreMesh(core_axis_name="c", subcore_axis_name="s")

    @functools.partial(
        pl.kernel,
        mesh=mesh,
        out_type=jax.ShapeDtypeStruct((2 * N, W), jnp.float32),
        compiler_params=pltpu.CompilerParams(use_tc_tiling_on_sc=False),
        scratch_types=[
            pltpu.VMEM((rw, _CHUNK), jnp.int32),
            pltpu.VMEM((_CHUNK, W), jnp.float32),
            pltpu.VMEM((rps, W), jnp.float32),
            pltpu.VMEM_SHARED((N, W), jnp.float32),
            pltpu.SemaphoreType.DMA,
        ],
    )
    def ker(dst_hbm, out_hbm, didx, ones, zbuf, acc, sem):
        c = lax.axis_index("c")
        s = lax.axis_index("s")
        w = c * _NS + s
        onehot = jnp.where(lax.iota(jnp.int32, 16) == 0,
                           jnp.float32(1), jnp.float32(0))
        zero16 = jnp.zeros((16,), jnp.float32)

        def fill(i, carry):
            ones[i, :] = onehot
            return carry

        lax.fori_loop(0, _CHUNK, fill, 0)

        def zrow(i, carry):
            zbuf[i, :] = zero16
            return carry

        lax.fori_loop(0, rps, zrow, 0)
        pltpu.sync_copy(zbuf, acc.at[pl.ds(s * rps, rps)])
        pltpu.sync_copy(dst_hbm.at[pl.ds(w * rw, rw)], didx)
        plsc.subcore_barrier()

        def body(g, carry):
            jb = g * _NBUF
            handles = [
                pltpu.async_copy(ones, acc.at[didx.at[jb + b]], sem, add=True)
                for b in range(_NBUF)
            ]
            for h in handles:
                h.wait()
            return carry

        lax.fori_loop(0, rw // _NBUF, body, 0)
        plsc.subcore_barrier()

        pltpu.sync_copy(acc.at[pl.ds(s * rps, rps)],
                        out_hbm.at[pl.ds(c * N + s * rps, rps)])

    return ker(dst2)


def _tc_matmul(x, W1):
    N = x.shape[0]
    H = W1.shape[1]

    def body(x_ref, w_ref, out_ref):
        out_ref[...] = jnp.dot(x_ref[...], w_ref[...],
                               preferred_element_type=jnp.float32)

    return pl.pallas_call(
        body,
        out_shape=jax.ShapeDtypeStruct((N, H), jnp.float32),
    )(x, W1)


def _tc_first(degp, xw):
    N, H = xw.shape

    def body(deg_ref, xw_ref, hp_ref, dinv_ref):
        deg = deg_ref[0:N, 0:1] + deg_ref[N:2 * N, 0:1] + 1.0
        dinv = lax.rsqrt(deg)
        dinv_ref[...] = dinv
        hp_ref[...] = xw_ref[...] * dinv

    return pl.pallas_call(
        body,
        out_shape=(jax.ShapeDtypeStruct((N, H), jnp.float32),
                   jax.ShapeDtypeStruct((N, 1), jnp.float32)),
    )(degp, xw)


def _tc_mid(e, hp, dinv, b, W):
    N, H = hp.shape
    HO = W.shape[1]

    def body(e_ref, hp_ref, dinv_ref, b_ref, w_ref, out_ref):
        esum = e_ref[0:N, :] + e_ref[N:2 * N, :] + hp_ref[...]
        h = jnp.maximum(esum * dinv_ref[...] + b_ref[...], 0.0)
        out_ref[...] = jnp.dot(h, w_ref[...],
                               preferred_element_type=jnp.float32) * dinv_ref[...]

    return pl.pallas_call(
        body,
        out_shape=jax.ShapeDtypeStruct((N, HO), jnp.float32),
    )(e, hp, dinv, b, W)


def _tc_final(e, hp, dinv, b3, Wc1, bc1, Wc2, bc2):
    N, H = hp.shape

    def body(e_ref, hp_ref, dinv_ref, b3_ref, wc1_ref, bc1_ref, wc2_ref,
             bc2_ref, out_ref):
        esum = e_ref[0:N, :] + e_ref[N:2 * N, :] + hp_ref[...]
        h3 = esum * dinv_ref[...] + b3_ref[...]
        g = jnp.sum(h3, axis=0, keepdims=True) * jnp.float32(1.0 / N)
        z = jnp.maximum(jnp.dot(g, wc1_ref[...],
                                preferred_element_type=jnp.float32)
                        + bc1_ref[...], 0.0)
        out_ref[...] = jnp.dot(z, wc2_ref[...],
                               preferred_element_type=jnp.float32) + bc2_ref[...]

    return pl.pallas_call(
        body,
        out_shape=jax.ShapeDtypeStruct((1, Wc2.shape[1]), jnp.float32),
    )(e, hp, dinv, b3, Wc1, bc1, Wc2, bc2)


def kernel(x, edge_index, W1, b1, W2, b2, W3, b3, Wc1, bc1, Wc2, bc2):
    N = x.shape[0]
    src2 = edge_index[0].reshape(-1, _CHUNK)
    dst2 = edge_index[1].reshape(-1, _CHUNK)

    xw1 = _tc_matmul(x, W1)
    degp = _sc_degree(dst2, N)
    h1p, dinv = _tc_first(degp, xw1)

    e1 = _sc_edge_accumulate(h1p, src2, dst2)
    h2p = _tc_mid(e1, h1p, dinv, b1.reshape(1, -1), W2)

    e2 = _sc_edge_accumulate(h2p, src2, dst2)
    h3p = _tc_mid(e2, h2p, dinv, b2.reshape(1, -1), W3)

    e3 = _sc_edge_accumulate(h3p, src2, dst2)
    out = _tc_final(e3, h3p, dinv, b3.reshape(1, -1), Wc1,
                    bc1.reshape(1, -1), Wc2, bc2.reshape(1, -1))
    return out


# speedup vs baseline: 43.1988x; 1.1523x over previous
import functools

import jax
import jax.numpy as jnp
from jax import lax
from jax.experimental import pallas as pl
from jax.experimental.pallas import tpu as pltpu
from jax.experimental.pallas import tpu_sc as plsc

_NC = 2
_NS = 16
_CHUNK = 100
_NBUF = 10
_ZROWS = 125


def _sc_edge_accumulate(hp, src2, dst2):
    N, H = hp.shape
    n_rows = src2.shape[0]
    nw = _NC * _NS
    rw = n_rows // nw
    n_grp = rw // _NBUF
    rps = N // _NS

    mesh = plsc.VectorSubcoreMesh(core_axis_name="c", subcore_axis_name="s")

    @functools.partial(
        pl.kernel,
        mesh=mesh,
        out_type=jax.ShapeDtypeStruct((2 * N, H), jnp.float32),
        compiler_params=pltpu.CompilerParams(use_tc_tiling_on_sc=False),
        scratch_types=[
            pltpu.VMEM((rw, _CHUNK), jnp.int32),
            pltpu.VMEM((rw, _CHUNK), jnp.int32),
            [pltpu.VMEM((_CHUNK, H), jnp.float32)] * _NBUF,
            pltpu.VMEM_SHARED((N, H), jnp.float32),
            pltpu.SemaphoreType.DMA,
        ],
    )
    def ker(hp_hbm, src_hbm, dst_hbm, out_hbm,
            sidx, didx, rows, acc, sem):
        c = lax.axis_index("c")
        s = lax.axis_index("s")
        w = c * _NS + s
        zero16 = jnp.zeros((16,), jnp.float32)

        def zrow(i, carry):
            for j in range(H // 16):
                rows[0][i, pl.ds(j * 16, 16)] = zero16
            return carry

        lax.fori_loop(0, _CHUNK, zrow, 0)
        zfull, zrem = divmod(rps, _CHUNK)
        for z in range(zfull):
            pltpu.sync_copy(rows[0], acc.at[pl.ds(s * rps + z * _CHUNK, _CHUNK)])
        if zrem:
            pltpu.sync_copy(rows[0].at[pl.ds(0, zrem)],
                            acc.at[pl.ds(s * rps + zfull * _CHUNK, zrem)])
        pltpu.sync_copy(src_hbm.at[pl.ds(w * rw, rw)], sidx)
        pltpu.sync_copy(dst_hbm.at[pl.ds(w * rw, rw)], didx)
        plsc.subcore_barrier()

        def group(g, carry):
            jb = g * _NBUF
            handles = [
                pltpu.async_copy(hp_hbm.at[sidx.at[jb + b]], rows[b], sem)
                for b in range(_NBUF)
            ]
            for b in range(_NBUF):
                handles[b].wait()
                pltpu.sync_copy(rows[b], acc.at[didx.at[jb + b]], add=True)
            return carry

        lax.fori_loop(0, n_grp, group, 0)
        plsc.subcore_barrier()

        pltpu.sync_copy(acc.at[pl.ds(s * rps, rps)],
                        out_hbm.at[pl.ds(c * N + s * rps, rps)])

    return ker(hp, src2, dst2)


def _sc_edge_accumulate_fused(hp, dinv16, src2, dst2):
    N, H = hp.shape
    W = dinv16.shape[1]
    n_rows = src2.shape[0]
    nw = _NC * _NS
    rw = n_rows // nw
    nbuf = 5
    n_grp = rw // nbuf
    rps = N // _NS

    mesh = plsc.VectorSubcoreMesh(core_axis_name="c", subcore_axis_name="s")

    @functools.partial(
        pl.kernel,
        mesh=mesh,
        out_type=(jax.ShapeDtypeStruct((2 * N, H), jnp.float32),
                  jax.ShapeDtypeStruct((2 * N, W), jnp.float32)),
        compiler_params=pltpu.CompilerParams(use_tc_tiling_on_sc=False),
        scratch_types=[
            pltpu.VMEM((rw, _CHUNK), jnp.int32),
            pltpu.VMEM((rw, _CHUNK), jnp.int32),
            [pltpu.VMEM((_CHUNK, H), jnp.float32)] * nbuf,
            [pltpu.VMEM((_CHUNK, W), jnp.float32)] * nbuf,
            pltpu.VMEM_SHARED((N, H), jnp.float32),
            pltpu.VMEM_SHARED((N, W), jnp.float32),
            pltpu.SemaphoreType.DMA,
            pltpu.SemaphoreType.DMA,
        ],
    )
    def ker(hp_hbm, dinv_hbm, src_hbm, dst_hbm, oute_hbm, outw_hbm,
            sidx, didx, rows, wrows, acce, accw, sem, semw):
        c = lax.axis_index("c")
        s = lax.axis_index("s")
        w = c * _NS + s
        zero16 = jnp.zeros((16,), jnp.float32)

        def zrow(i, carry):
            for j in range(H // 16):
                rows[0][i, pl.ds(j * 16, 16)] = zero16
            wrows[0][i, :] = zero16
            return carry

        lax.fori_loop(0, _CHUNK, zrow, 0)
        zfull, zrem = divmod(rps, _CHUNK)
        for z in range(zfull):
            pltpu.sync_copy(rows[0], acce.at[pl.ds(s * rps + z * _CHUNK, _CHUNK)])
            pltpu.sync_copy(wrows[0], accw.at[pl.ds(s * rps + z * _CHUNK, _CHUNK)])
        if zrem:
            pltpu.sync_copy(rows[0].at[pl.ds(0, zrem)],
                            acce.at[pl.ds(s * rps + zfull * _CHUNK, zrem)])
            pltpu.sync_copy(wrows[0].at[pl.ds(0, zrem)],
                            accw.at[pl.ds(s * rps + zfull * _CHUNK, zrem)])
        pltpu.sync_copy(src_hbm.at[pl.ds(w * rw, rw)], sidx)
        pltpu.sync_copy(dst_hbm.at[pl.ds(w * rw, rw)], didx)
        plsc.subcore_barrier()

        def group(g, carry):
            jb = g * nbuf
            eh = [pltpu.async_copy(hp_hbm.at[sidx.at[jb + b]], rows[b], sem)
                  for b in range(nbuf)]
            wh = [pltpu.async_copy(dinv_hbm.at[didx.at[jb + b]], wrows[b], semw)
                  for b in range(nbuf)]
            for b in range(nbuf):
                eh[b].wait()
                pltpu.sync_copy(rows[b], acce.at[didx.at[jb + b]], add=True)
                wh[b].wait()
                pltpu.sync_copy(wrows[b], accw.at[sidx.at[jb + b]], add=True)
            return carry

        lax.fori_loop(0, n_grp, group, 0)
        plsc.subcore_barrier()

        pltpu.sync_copy(acce.at[pl.ds(s * rps, rps)],
                        oute_hbm.at[pl.ds(c * N + s * rps, rps)])
        pltpu.sync_copy(accw.at[pl.ds(s * rps, rps)],
                        outw_hbm.at[pl.ds(c * N + s * rps, rps)])

    return ker(hp, dinv16, src2, dst2)


def _sc_degree(dst2, n_nodes):
    N = n_nodes
    W = 16
    n_rows = dst2.shape[0]
    nw = _NC * _NS
    rw = n_rows // nw
    rps = N // _NS

    mesh = plsc.VectorSubcoreMesh(core_axis_name="c", subcore_axis_name="s")

    @functools.partial(
        pl.kernel,
        mesh=mesh,
        out_type=jax.ShapeDtypeStruct((2 * N, W), jnp.float32),
        compiler_params=pltpu.CompilerParams(use_tc_tiling_on_sc=False),
        scratch_types=[
            pltpu.VMEM((rw, _CHUNK), jnp.int32),
            pltpu.VMEM((_CHUNK, W), jnp.float32),
            pltpu.VMEM((rps, W), jnp.float32),
            pltpu.VMEM_SHARED((N, W), jnp.float32),
            pltpu.SemaphoreType.DMA,
        ],
    )
    def ker(dst_hbm, out_hbm, didx, ones, zbuf, acc, sem):
        c = lax.axis_index("c")
        s = lax.axis_index("s")
        w = c * _NS + s
        onehot = jnp.where(lax.iota(jnp.int32, 16) == 0,
                           jnp.float32(1), jnp.float32(0))
        zero16 = jnp.zeros((16,), jnp.float32)

        def fill(i, carry):
            ones[i, :] = onehot
            return carry

        lax.fori_loop(0, _CHUNK, fill, 0)

        def zrow(i, carry):
            zbuf[i, :] = zero16
            return carry

        lax.fori_loop(0, rps, zrow, 0)
        pltpu.sync_copy(zbuf, acc.at[pl.ds(s * rps, rps)])
        pltpu.sync_copy(dst_hbm.at[pl.ds(w * rw, rw)], didx)
        plsc.subcore_barrier()

        def body(g, carry):
            jb = g * _NBUF
            handles = [
                pltpu.async_copy(ones, acc.at[didx.at[jb + b]], sem, add=True)
                for b in range(_NBUF)
            ]
            for h in handles:
                h.wait()
            return carry

        lax.fori_loop(0, rw // _NBUF, body, 0)
        plsc.subcore_barrier()

        pltpu.sync_copy(acc.at[pl.ds(s * rps, rps)],
                        out_hbm.at[pl.ds(c * N + s * rps, rps)])

    return ker(dst2)


def _tc_matmul(x, W1):
    N = x.shape[0]
    H = W1.shape[1]

    def body(x_ref, w_ref, out_ref):
        out_ref[...] = jnp.dot(x_ref[...], w_ref[...],
                               preferred_element_type=jnp.float32)

    return pl.pallas_call(
        body,
        out_shape=jax.ShapeDtypeStruct((N, H), jnp.float32),
    )(x, W1)


def _tc_first(degp, xw):
    N, H = xw.shape

    def body(deg_ref, xw_ref, hp_ref, dinv_ref, dinv16_ref):
        deg = deg_ref[0:N, 0:1] + deg_ref[N:2 * N, 0:1] + 1.0
        dinv = lax.rsqrt(deg)
        dinv_ref[...] = dinv
        dinv16_ref[...] = jnp.broadcast_to(dinv, (N, 16))
        hp_ref[...] = xw_ref[...] * dinv

    return pl.pallas_call(
        body,
        out_shape=(jax.ShapeDtypeStruct((N, H), jnp.float32),
                   jax.ShapeDtypeStruct((N, 1), jnp.float32),
                   jax.ShapeDtypeStruct((N, 16), jnp.float32)),
    )(degp, xw)


def _tc_mid(e, hp, dinv, b, W):
    N, H = hp.shape
    HO = W.shape[1]

    def body(e_ref, hp_ref, dinv_ref, b_ref, w_ref, out_ref):
        esum = e_ref[0:N, :] + e_ref[N:2 * N, :] + hp_ref[...]
        h = jnp.maximum(esum * dinv_ref[...] + b_ref[...], 0.0)
        out_ref[...] = jnp.dot(h, w_ref[...],
                               preferred_element_type=jnp.float32) * dinv_ref[...]

    return pl.pallas_call(
        body,
        out_shape=jax.ShapeDtypeStruct((N, HO), jnp.float32),
    )(e, hp, dinv, b, W)


def _tc_final(wsump, hp, dinv, b3, Wc1, bc1, Wc2, bc2):
    N, H = hp.shape

    def body(w_ref, hp_ref, dinv_ref, b3_ref, wc1_ref, bc1_ref, wc2_ref,
             bc2_ref, out_ref):
        wt = w_ref[0:N, 0:1] + w_ref[N:2 * N, 0:1] + dinv_ref[...]
        g = (jnp.sum(hp_ref[...] * wt, axis=0, keepdims=True)
             * jnp.float32(1.0 / N) + b3_ref[...])
        z = jnp.maximum(jnp.dot(g, wc1_ref[...],
                                preferred_element_type=jnp.float32)
                        + bc1_ref[...], 0.0)
        out_ref[...] = jnp.dot(z, wc2_ref[...],
                               preferred_element_type=jnp.float32) + bc2_ref[...]

    return pl.pallas_call(
        body,
        out_shape=jax.ShapeDtypeStruct((1, Wc2.shape[1]), jnp.float32),
    )(wsump, hp, dinv, b3, Wc1, bc1, Wc2, bc2)


def kernel(x, edge_index, W1, b1, W2, b2, W3, b3, Wc1, bc1, Wc2, bc2):
    N = x.shape[0]
    src2 = edge_index[0].reshape(-1, _CHUNK)
    dst2 = edge_index[1].reshape(-1, _CHUNK)

    xw1 = _tc_matmul(x, W1)
    degp = _sc_degree(dst2, N)
    h1p, dinv, dinv16 = _tc_first(degp, xw1)

    e1, wsump = _sc_edge_accumulate_fused(h1p, dinv16, src2, dst2)
    h2p = _tc_mid(e1, h1p, dinv, b1.reshape(1, -1), W2)

    e2 = _sc_edge_accumulate(h2p, src2, dst2)
    h3p = _tc_mid(e2, h2p, dinv, b2.reshape(1, -1), W3)

    out = _tc_final(wsump, h3p, dinv, b3.reshape(1, -1), Wc1,
                    bc1.reshape(1, -1), Wc2, bc2.reshape(1, -1))
    return out


# speedup vs baseline: 44.1116x; 1.0211x over previous
import functools

import jax
import jax.numpy as jnp
from jax import lax
from jax.experimental import pallas as pl
from jax.experimental.pallas import tpu as pltpu
from jax.experimental.pallas import tpu_sc as plsc

_NC = 2
_NS = 16
_CHUNK = 100
_NBUF = 10
_ZROWS = 125


def _sc_edge_accumulate(hp, src2, dst2):
    N, H = hp.shape
    n_rows = src2.shape[0]
    nw = _NC * _NS
    rw = n_rows // nw
    n_grp = rw // _NBUF
    rps = N // _NS

    mesh = plsc.VectorSubcoreMesh(core_axis_name="c", subcore_axis_name="s")

    @functools.partial(
        pl.kernel,
        mesh=mesh,
        out_type=jax.ShapeDtypeStruct((2 * N, H), jnp.float32),
        compiler_params=pltpu.CompilerParams(use_tc_tiling_on_sc=False),
        scratch_types=[
            pltpu.VMEM((rw, _CHUNK), jnp.int32),
            pltpu.VMEM((rw, _CHUNK), jnp.int32),
            [pltpu.VMEM((_CHUNK, H), jnp.float32)] * _NBUF,
            pltpu.VMEM_SHARED((N, H), jnp.float32),
            pltpu.SemaphoreType.DMA,
        ],
    )
    def ker(hp_hbm, src_hbm, dst_hbm, out_hbm,
            sidx, didx, rows, acc, sem):
        c = lax.axis_index("c")
        s = lax.axis_index("s")
        w = c * _NS + s
        zero16 = jnp.zeros((16,), jnp.float32)

        def zrow(i, carry):
            for j in range(H // 16):
                rows[0][i, pl.ds(j * 16, 16)] = zero16
            return carry

        lax.fori_loop(0, _CHUNK, zrow, 0)
        zfull, zrem = divmod(rps, _CHUNK)
        for z in range(zfull):
            pltpu.sync_copy(rows[0], acc.at[pl.ds(s * rps + z * _CHUNK, _CHUNK)])
        if zrem:
            pltpu.sync_copy(rows[0].at[pl.ds(0, zrem)],
                            acc.at[pl.ds(s * rps + zfull * _CHUNK, zrem)])
        pltpu.sync_copy(src_hbm.at[pl.ds(w * rw, rw)], sidx)
        pltpu.sync_copy(dst_hbm.at[pl.ds(w * rw, rw)], didx)
        plsc.subcore_barrier()

        def group(g, carry):
            jb = g * _NBUF
            handles = [
                pltpu.async_copy(hp_hbm.at[sidx.at[jb + b]], rows[b], sem)
                for b in range(_NBUF)
            ]
            for b in range(_NBUF):
                handles[b].wait()
                pltpu.sync_copy(rows[b], acc.at[didx.at[jb + b]], add=True)
            return carry

        lax.fori_loop(0, n_grp, group, 0)
        plsc.subcore_barrier()

        pltpu.sync_copy(acc.at[pl.ds(s * rps, rps)],
                        out_hbm.at[pl.ds(c * N + s * rps, rps)])

    return ker(hp, src2, dst2)


def _sc_edge_accumulate_fused(hp, dinv16, src2, dst2):
    N, H = hp.shape
    W = dinv16.shape[1]
    n_rows = src2.shape[0]
    nw = _NC * _NS
    rw = n_rows // nw
    nbuf = 5
    n_grp = rw // nbuf
    rps = N // _NS

    mesh = plsc.VectorSubcoreMesh(core_axis_name="c", subcore_axis_name="s")

    @functools.partial(
        pl.kernel,
        mesh=mesh,
        out_type=(jax.ShapeDtypeStruct((2 * N, H), jnp.float32),
                  jax.ShapeDtypeStruct((2 * N, W), jnp.float32)),
        compiler_params=pltpu.CompilerParams(use_tc_tiling_on_sc=False),
        scratch_types=[
            pltpu.VMEM((rw, _CHUNK), jnp.int32),
            pltpu.VMEM((rw, _CHUNK), jnp.int32),
            [pltpu.VMEM((_CHUNK, H), jnp.float32)] * nbuf,
            [pltpu.VMEM((_CHUNK, W), jnp.float32)] * nbuf,
            pltpu.VMEM_SHARED((N, H), jnp.float32),
            pltpu.VMEM_SHARED((N, W), jnp.float32),
            pltpu.SemaphoreType.DMA,
            pltpu.SemaphoreType.DMA,
        ],
    )
    def ker(hp_hbm, dinv_hbm, src_hbm, dst_hbm, oute_hbm, outw_hbm,
            sidx, didx, rows, wrows, acce, accw, sem, semw):
        c = lax.axis_index("c")
        s = lax.axis_index("s")
        w = c * _NS + s
        zero16 = jnp.zeros((16,), jnp.float32)

        def zrow(i, carry):
            for j in range(H // 16):
                rows[0][i, pl.ds(j * 16, 16)] = zero16
            wrows[0][i, :] = zero16
            return carry

        lax.fori_loop(0, _CHUNK, zrow, 0)
        zfull, zrem = divmod(rps, _CHUNK)
        for z in range(zfull):
            pltpu.sync_copy(rows[0], acce.at[pl.ds(s * rps + z * _CHUNK, _CHUNK)])
            pltpu.sync_copy(wrows[0], accw.at[pl.ds(s * rps + z * _CHUNK, _CHUNK)])
        if zrem:
            pltpu.sync_copy(rows[0].at[pl.ds(0, zrem)],
                            acce.at[pl.ds(s * rps + zfull * _CHUNK, zrem)])
            pltpu.sync_copy(wrows[0].at[pl.ds(0, zrem)],
                            accw.at[pl.ds(s * rps + zfull * _CHUNK, zrem)])
        pltpu.sync_copy(src_hbm.at[pl.ds(w * rw, rw)], sidx)
        pltpu.sync_copy(dst_hbm.at[pl.ds(w * rw, rw)], didx)
        plsc.subcore_barrier()

        def group(g, carry):
            jb = g * nbuf
            eh = [pltpu.async_copy(hp_hbm.at[sidx.at[jb + b]], rows[b], sem)
                  for b in range(nbuf)]
            wh = [pltpu.async_copy(dinv_hbm.at[didx.at[jb + b]], wrows[b], semw)
                  for b in range(nbuf)]
            for b in range(nbuf):
                eh[b].wait()
                pltpu.sync_copy(rows[b], acce.at[didx.at[jb + b]], add=True)
                wh[b].wait()
                pltpu.sync_copy(wrows[b], accw.at[sidx.at[jb + b]], add=True)
            return carry

        lax.fori_loop(0, n_grp, group, 0)
        plsc.subcore_barrier()

        pltpu.sync_copy(acce.at[pl.ds(s * rps, rps)],
                        oute_hbm.at[pl.ds(c * N + s * rps, rps)])
        pltpu.sync_copy(accw.at[pl.ds(s * rps, rps)],
                        outw_hbm.at[pl.ds(c * N + s * rps, rps)])

    return ker(hp, dinv16, src2, dst2)


def _sc_degree(dst2, n_nodes):
    N = n_nodes
    W = 16
    n_rows = dst2.shape[0]
    nw = _NC * _NS
    rw = n_rows // nw
    rps = N // _NS

    mesh = plsc.VectorSubcoreMesh(core_axis_name="c", subcore_axis_name="s")

    @functools.partial(
        pl.kernel,
        mesh=mesh,
        out_type=jax.ShapeDtypeStruct((2 * N, W), jnp.float32),
        compiler_params=pltpu.CompilerParams(use_tc_tiling_on_sc=False),
        scratch_types=[
            pltpu.VMEM((rw, _CHUNK), jnp.int32),
            pltpu.VMEM((_CHUNK, W), jnp.float32),
            pltpu.VMEM((rps, W), jnp.float32),
            pltpu.VMEM_SHARED((N, W), jnp.float32),
            pltpu.SemaphoreType.DMA,
        ],
    )
    def ker(dst_hbm, out_hbm, didx, ones, zbuf, acc, sem):
        c = lax.axis_index("c")
        s = lax.axis_index("s")
        w = c * _NS + s
        onehot = jnp.where(lax.iota(jnp.int32, 16) == 0,
                           jnp.float32(1), jnp.float32(0))
        zero16 = jnp.zeros((16,), jnp.float32)

        def fill(i, carry):
            ones[i, :] = onehot
            return carry

        lax.fori_loop(0, _CHUNK, fill, 0)

        def zrow(i, carry):
            zbuf[i, :] = zero16
            return carry

        lax.fori_loop(0, rps, zrow, 0)
        pltpu.sync_copy(zbuf, acc.at[pl.ds(s * rps, rps)])
        pltpu.sync_copy(dst_hbm.at[pl.ds(w * rw, rw)], didx)
        plsc.subcore_barrier()

        def body(g, carry):
            jb = g * _NBUF
            handles = [
                pltpu.async_copy(ones, acc.at[didx.at[jb + b]], sem, add=True)
                for b in range(_NBUF)
            ]
            for h in handles:
                h.wait()
            return carry

        lax.fori_loop(0, rw // _NBUF, body, 0)
        plsc.subcore_barrier()

        pltpu.sync_copy(acc.at[pl.ds(s * rps, rps)],
                        out_hbm.at[pl.ds(c * N + s * rps, rps)])

    return ker(dst2)


def _tc_matmul(x, W1):
    N = x.shape[0]
    H = W1.shape[1]

    def body(x_ref, w_ref, out_ref):
        out_ref[...] = jnp.dot(x_ref[...], w_ref[...],
                               preferred_element_type=jnp.float32)

    return pl.pallas_call(
        body,
        out_shape=jax.ShapeDtypeStruct((N, H), jnp.float32),
    )(x, W1)


def _tc_first(degp, xw):
    N, H = xw.shape

    def body(deg_ref, xw_ref, hp_ref, dinv_ref, dinv16_ref):
        deg = deg_ref[0:N, 0:1] + deg_ref[N:2 * N, 0:1] + 1.0
        dinv = lax.rsqrt(deg)
        dinv_ref[...] = dinv
        dinv16_ref[...] = jnp.broadcast_to(dinv, (N, 16))
        hp_ref[...] = xw_ref[...] * dinv

    return pl.pallas_call(
        body,
        out_shape=(jax.ShapeDtypeStruct((N, H), jnp.float32),
                   jax.ShapeDtypeStruct((N, 1), jnp.float32),
                   jax.ShapeDtypeStruct((N, 16), jnp.float32)),
    )(degp, xw)


def _tc_mid(e, hp, dinv, b, W):
    N, H = hp.shape
    HO = W.shape[1]

    def body(e_ref, hp_ref, dinv_ref, b_ref, w_ref, out_ref):
        esum = e_ref[0:N, :] + e_ref[N:2 * N, :] + hp_ref[...]
        h = jnp.maximum(esum * dinv_ref[...] + b_ref[...], 0.0)
        out_ref[...] = jnp.dot(h, w_ref[...],
                               preferred_element_type=jnp.float32) * dinv_ref[...]

    return pl.pallas_call(
        body,
        out_shape=jax.ShapeDtypeStruct((N, HO), jnp.float32),
    )(e, hp, dinv, b, W)


def _tc_final(e, hp, dinv, b2, W3, wsump, b3, Wc1, bc1, Wc2, bc2):
    N, H = hp.shape

    def body(e_ref, hp_ref, dinv_ref, b2_ref, w3_ref, w_ref, b3_ref,
             wc1_ref, bc1_ref, wc2_ref, bc2_ref, out_ref):
        dinv = dinv_ref[...]
        esum = e_ref[0:N, :] + e_ref[N:2 * N, :] + hp_ref[...]
        h2 = jnp.maximum(esum * dinv + b2_ref[...], 0.0)
        v = (w_ref[0:N, 0:1] + w_ref[N:2 * N, 0:1] + dinv) * dinv
        u = jnp.sum(h2 * v, axis=0, keepdims=True) * jnp.float32(1.0 / N)
        g = jnp.dot(u, w3_ref[...],
                    preferred_element_type=jnp.float32) + b3_ref[...]
        z = jnp.maximum(jnp.dot(g, wc1_ref[...],
                                preferred_element_type=jnp.float32)
                        + bc1_ref[...], 0.0)
        out_ref[...] = jnp.dot(z, wc2_ref[...],
                               preferred_element_type=jnp.float32) + bc2_ref[...]

    return pl.pallas_call(
        body,
        out_shape=jax.ShapeDtypeStruct((1, Wc2.shape[1]), jnp.float32),
    )(e, hp, dinv, b2, W3, wsump, b3, Wc1, bc1, Wc2, bc2)


def kernel(x, edge_index, W1, b1, W2, b2, W3, b3, Wc1, bc1, Wc2, bc2):
    N = x.shape[0]
    src2 = edge_index[0].reshape(-1, _CHUNK)
    dst2 = edge_index[1].reshape(-1, _CHUNK)

    xw1 = _tc_matmul(x, W1)
    degp = _sc_degree(dst2, N)
    h1p, dinv, dinv16 = _tc_first(degp, xw1)

    e1, wsump = _sc_edge_accumulate_fused(h1p, dinv16, src2, dst2)
    h2p = _tc_mid(e1, h1p, dinv, b1.reshape(1, -1), W2)

    e2 = _sc_edge_accumulate(h2p, src2, dst2)
    out = _tc_final(e2, h2p, dinv, b2.reshape(1, -1), W3, wsump,
                    b3.reshape(1, -1), Wc1, bc1.reshape(1, -1), Wc2,
                    bc2.reshape(1, -1))
    return out
